# baseline (speedup 1.0000x reference)
import functools

import jax
import jax.numpy as jnp
from jax import lax
from jax.experimental import pallas as pl
from jax.experimental.pallas import tpu as pltpu
from jax.experimental.pallas import tpu_sc as plsc

_NUM_CATEGORIES = 1000
_TOKEN_DIM = 768
_BATCH = 4096
_SC_ROWS = 1024
_BT = 1024


@functools.cache
def _build_sc():
    info = plsc.get_sparse_core_info()
    nc, ns = info.num_cores, info.num_subcores
    nw = nc * ns
    b_per_w = _SC_ROWS // nw

    mesh = plsc.VectorSubcoreMesh(core_axis_name="c", subcore_axis_name="s")

    @functools.partial(
        pl.kernel,
        mesh=mesh,
        out_type=jax.ShapeDtypeStruct((_SC_ROWS, _TOKEN_DIM), jnp.float32),
        scratch_types=[
            pltpu.VMEM((b_per_w,), jnp.int32),
            pltpu.VMEM((b_per_w, _TOKEN_DIM), jnp.float32),
            pltpu.SemaphoreType.DMA,
        ],
    )
    def gather(table_hbm, idx_hbm, out_hbm, idx_v, rows_v, sem):
        wid = lax.axis_index("s") * nc + lax.axis_index("c")
        base = wid * b_per_w
        pltpu.sync_copy(idx_hbm.at[pl.ds(base, b_per_w)], idx_v)
        pltpu.async_copy(table_hbm.at[idx_v], rows_v, sem).wait()
        pltpu.sync_copy(rows_v, out_hbm.at[pl.ds(base, b_per_w)])

    return gather


def _tc_body(idx_ref, table_ref, out_ref, tbf_ref):
    @pl.when(pl.program_id(0) == 0)
    def _():
        tbf_ref[...] = table_ref[...].astype(jnp.bfloat16)

    k = lax.broadcasted_iota(jnp.int32, (_NUM_CATEGORIES, _BT), 0)
    oh = (k == idx_ref[...][None, :]).astype(jnp.bfloat16)
    out_ref[...] = lax.dot_general(
        oh,
        tbf_ref[...],
        (((0,), (0,)), ((), ())),
        preferred_element_type=jnp.float32,
    )


@functools.cache
def _build_tc():
    tc_rows = _BATCH - _SC_ROWS
    grid = tc_rows // _BT
    return pl.pallas_call(
        _tc_body,
        grid=(grid,),
        in_specs=[
            pl.BlockSpec((_BT,), lambda i: (i,)),
            pl.BlockSpec((_NUM_CATEGORIES, _TOKEN_DIM), lambda i: (0, 0)),
        ],
        out_specs=pl.BlockSpec((_BT, _TOKEN_DIM), lambda i: (i, 0)),
        out_shape=jax.ShapeDtypeStruct((tc_rows, _TOKEN_DIM), jnp.float32),
        scratch_shapes=[pltpu.VMEM((_NUM_CATEGORIES, _TOKEN_DIM), jnp.bfloat16)],
    )


def kernel(categories, virtual_tokens):
    idx = categories.astype(jnp.int32)
    sc_out = _build_sc()(virtual_tokens, idx[:_SC_ROWS])
    tc_out = _build_tc()(idx[_SC_ROWS:], virtual_tokens)
    return jnp.concatenate([sc_out, tc_out], axis=0)


# speedup vs baseline: 1.2606x; 1.2606x over previous
import functools

import jax
import jax.numpy as jnp
from jax import lax
from jax.experimental import pallas as pl
from jax.experimental.pallas import tpu as pltpu
from jax.experimental.pallas import tpu_sc as plsc

_NUM_CATEGORIES = 1000
_TOKEN_DIM = 768
_BATCH = 4096
_SC_ROWS = 1024
_BT = 1024


@functools.cache
def _build_sc():
    info = plsc.get_sparse_core_info()
    nc, ns = info.num_cores, info.num_subcores
    nw = nc * ns
    b_per_w = _SC_ROWS // nw

    mesh = plsc.VectorSubcoreMesh(core_axis_name="c", subcore_axis_name="s")

    @functools.partial(
        pl.kernel,
        mesh=mesh,
        out_type=jax.ShapeDtypeStruct((_SC_ROWS, _TOKEN_DIM), jnp.float32),
        scratch_types=[
            pltpu.VMEM((b_per_w,), jnp.int32),
            pltpu.VMEM((b_per_w, _TOKEN_DIM), jnp.float32),
            pltpu.SemaphoreType.DMA,
        ],
    )
    def gather(table_hbm, idx_hbm, out_hbm, idx_v, rows_v, sem):
        wid = lax.axis_index("s") * nc + lax.axis_index("c")
        base = wid * b_per_w
        pltpu.sync_copy(idx_hbm.at[pl.ds(base, b_per_w)], idx_v)
        pltpu.async_copy(table_hbm.at[idx_v], rows_v, sem).wait()
        pltpu.sync_copy(rows_v, out_hbm.at[pl.ds(base, b_per_w)])

    return gather


def _tc_body(idx_ref, table_ref, out_ref, tbf_ref):
    @pl.when(pl.program_id(0) == 0)
    def _():
        tbf_ref[...] = table_ref[...].astype(jnp.bfloat16)

    k = lax.broadcasted_iota(jnp.int32, (_NUM_CATEGORIES, _BT), 0)
    oh = (k == idx_ref[...][None, :]).astype(jnp.bfloat16)
    out_ref[...] = lax.dot_general(
        oh,
        tbf_ref[...],
        (((0,), (0,)), ((), ())),
        preferred_element_type=jnp.float32,
    )


_SC_BLOCKS = _SC_ROWS // _BT


@functools.cache
def _build_tc():
    grid = (_BATCH - _SC_ROWS) // _BT
    return pl.pallas_call(
        _tc_body,
        grid=(grid,),
        in_specs=[
            pl.BlockSpec((_BT,), lambda i: (i + _SC_BLOCKS,)),
            pl.BlockSpec((_NUM_CATEGORIES, _TOKEN_DIM), lambda i: (0, 0)),
        ],
        out_specs=pl.BlockSpec((_BT, _TOKEN_DIM), lambda i: (i + _SC_BLOCKS, 0)),
        out_shape=jax.ShapeDtypeStruct((_BATCH, _TOKEN_DIM), jnp.float32),
        scratch_shapes=[pltpu.VMEM((_NUM_CATEGORIES, _TOKEN_DIM), jnp.bfloat16)],
    )


def kernel(categories, virtual_tokens):
    idx = categories.astype(jnp.int32)
    sc_out = _build_sc()(virtual_tokens, idx)
    tc_out = _build_tc()(idx, virtual_tokens)
    return lax.dynamic_update_slice(tc_out, sc_out, (0, 0))
